# baseline (speedup 1.0000x reference)
import functools

import jax
import jax.numpy as jnp
from jax import lax
from jax.experimental import pallas as pl
from jax.experimental.pallas import tpu as pltpu
from jax.experimental.pallas import tpu_sc as plsc

VOCAB = 1000000
D = 32
BATCH = 16384
HIST = 50
L = 16
NW = 32

TOTAL = BATCH * HIST
CHUNK = 128
ROWS = TOTAL // CHUNK
ROWS_PER_W = ROWS // NW
GROUP = 8
NGROUPS = ROWS_PER_W // GROUP
GROUP_ROWS = GROUP * CHUNK

CBLOCKS = (VOCAB + 127) // 128
VPAD = CBLOCKS * 128
CJ = ((CBLOCKS + NW - 1) // NW + 1) // 2 * 2


def _make_relayout():
  mesh = plsc.VectorSubcoreMesh(core_axis_name="c", subcore_axis_name="s")

  @functools.partial(
      pl.kernel,
      out_type=jax.ShapeDtypeStruct((VPAD * D,), jnp.float32),
      mesh=mesh,
      scratch_types=[
          pltpu.VMEM((D, 128), jnp.float32),
          pltpu.VMEM((D, 128), jnp.float32),
          pltpu.VMEM((128 * D,), jnp.float32),
          pltpu.VMEM((128 * D,), jnp.float32),
          pltpu.SemaphoreType.DMA,
          pltpu.SemaphoreType.DMA,
      ],
      compiler_params=pltpu.CompilerParams(
          use_tc_tiling_on_sc=True, disable_bounds_checks=True,
          needs_layout_passes=False),
  )
  def relayout_kernel(tab_hbm, r_hbm, inb0, inb1, outb0, outb1, isem, osem):
    wid = lax.axis_index("s") * 2 + lax.axis_index("c")
    iota = lax.iota(jnp.int32, L)
    bufs = ((inb0, outb0), (inb1, outb1))

    def do_block(q, inb, outb):
      pltpu.async_copy(
          tab_hbm.at[:, pl.ds(q * 128, 128)], inb, isem).wait()

      for vl in range(128):
        cidx = jnp.full((L,), vl, jnp.int32)
        for half in range(2):
          vec = plsc.load_gather(inb, [iota + half * L, cidx])
          outb[pl.ds(vl * D + half * L, L)] = vec

      pltpu.async_copy(
          outb, r_hbm.at[pl.ds(q * 128 * D, 128 * D)], osem)

    def pair_body(p, carry):
      for sub in range(2):
        j = p * 2 + sub
        q = wid + NW * j
        inb, outb = bufs[sub]

        @pl.when(q < CBLOCKS)
        def _():
          @pl.when(j >= 2)
          def _():
            pltpu.make_async_copy(
                outb, r_hbm.at[pl.ds(0, 128 * D)], osem).wait()
          do_block(q, inb, outb)
      return carry

    lax.fori_loop(0, CJ // 2, pair_body, 0)
    for _ in range(2):
      pltpu.make_async_copy(
          outb0, r_hbm.at[pl.ds(0, 128 * D)], osem).wait()

  return relayout_kernel


def _make_gather():
  mesh = plsc.VectorSubcoreMesh(core_axis_name="c", subcore_axis_name="s")

  @functools.partial(
      pl.kernel,
      out_type=jax.ShapeDtypeStruct((TOTAL, D), jnp.float32),
      mesh=mesh,
      scratch_types=[
          pltpu.VMEM((ROWS_PER_W, CHUNK), jnp.int32),
          pltpu.VMEM((2, GROUP_ROWS, D), jnp.float32),
          pltpu.SemaphoreType.DMA,
          pltpu.SemaphoreType.DMA,
      ],
      compiler_params=pltpu.CompilerParams(use_tc_tiling_on_sc=False),
  )
  def gather_kernel(table_hbm, idx_hbm, out_hbm, idx_v, rows_v, gsem, osem):
    wid = lax.axis_index("s") * 2 + lax.axis_index("c")
    base = wid * ROWS_PER_W
    pltpu.sync_copy(idx_hbm.at[pl.ds(base, ROWS_PER_W)], idx_v)

    def group_body(g, carry):
      buf = lax.rem(g, 2)

      @pl.when(g >= 2)
      def _():
        pltpu.make_async_copy(
            rows_v.at[buf], out_hbm.at[pl.ds(0, GROUP_ROWS)], osem).wait()

      for b in range(GROUP):
        pltpu.async_copy(
            table_hbm.at[idx_v.at[g * GROUP + b]],
            rows_v.at[buf, pl.ds(b * CHUNK, CHUNK)], gsem)
      for b in range(GROUP):
        pltpu.make_async_copy(
            table_hbm.at[idx_v.at[g * GROUP + b]],
            rows_v.at[buf, pl.ds(b * CHUNK, CHUNK)], gsem).wait()

      pltpu.async_copy(
          rows_v.at[buf],
          out_hbm.at[pl.ds((base + g * GROUP) * CHUNK, GROUP_ROWS)], osem)
      return carry

    lax.fori_loop(0, NGROUPS, group_body, 0)
    for _ in range(2):
      pltpu.make_async_copy(
          rows_v.at[0], out_hbm.at[pl.ds(0, GROUP_ROWS)], osem).wait()

  return gather_kernel


def _make_tiler():
  mesh = plsc.VectorSubcoreMesh(core_axis_name="c", subcore_axis_name="s")
  nq_per_w = (HIST * (BATCH // 128)) // NW

  @functools.partial(
      pl.kernel,
      out_type=jax.ShapeDtypeStruct((HIST, D, BATCH), jnp.float32),
      mesh=mesh,
      scratch_types=[
          pltpu.VMEM((128 * D,), jnp.float32),
          pltpu.VMEM((128 * D,), jnp.float32),
          pltpu.VMEM((1, D, 128), jnp.float32),
          pltpu.VMEM((1, D, 128), jnp.float32),
          pltpu.SemaphoreType.DMA,
          pltpu.SemaphoreType.DMA,
      ],
      compiler_params=pltpu.CompilerParams(
          use_tc_tiling_on_sc=True, needs_layout_passes=False),
  )
  def tiler_kernel(flat_hbm, out_hbm, inb0, inb1, outb0, outb1, isem, osem):
    wid = lax.axis_index("s") * 2 + lax.axis_index("c")
    iota_d = lax.iota(jnp.int32, L) * D
    bufs = ((inb0, outb0), (inb1, outb1))

    def do_chunk(q, inb, outb):
      pltpu.async_copy(
          flat_hbm.at[pl.ds(q * 128 * D, 128 * D)], inb, isem).wait()
      for d in range(D):
        for k in range(8):
          vec = plsc.load_gather(inb, [iota_d + (k * L * D + d)])
          outb[0, d, pl.ds(k * L, L)] = vec
      h = q // 128
      cb = lax.rem(q, 128)
      pltpu.async_copy(
          outb, out_hbm.at[pl.ds(h, 1), :, pl.ds(cb * 128, 128)], osem)

    def pair_body(p, carry):
      for sub in range(2):
        i = p * 2 + sub
        inb, outb = bufs[sub]

        @pl.when(i >= 2)
        def _():
          pltpu.make_async_copy(
              outb, out_hbm.at[pl.ds(0, 1), :, pl.ds(0, 128)], osem).wait()
        do_chunk(wid * nq_per_w + i, inb, outb)
      return carry

    lax.fori_loop(0, nq_per_w // 2, pair_body, 0)
    for _ in range(2):
      pltpu.make_async_copy(
          outb0, out_hbm.at[pl.ds(0, 1), :, pl.ds(0, 128)], osem).wait()

  return tiler_kernel


_relayout = _make_relayout()
_gather = _make_gather()
_tiler = _make_tiler()


@jax.jit
def kernel(x, embeddings):
  tab_t = jnp.transpose(embeddings)
  r_flat = _relayout(tab_t)
  r2d = r_flat.reshape(VPAD, D)
  idx2d = jnp.transpose(x).astype(jnp.int32).reshape(ROWS, CHUNK)
  rows = _gather(r2d, idx2d)
  flat = rows.reshape(TOTAL * D)
  out_t = _tiler(flat)
  return jnp.transpose(out_t, (2, 0, 1))


# speedup vs baseline: 1.2798x; 1.2798x over previous
import functools

import jax
import jax.numpy as jnp
from jax import lax
from jax.experimental import pallas as pl
from jax.experimental.pallas import tpu as pltpu
from jax.experimental.pallas import tpu_sc as plsc

VOCAB = 1000000
D = 32
BATCH = 16384
HIST = 50
L = 16
NW = 32

TOTAL = BATCH * HIST
CHUNK = 128
ROWS = TOTAL // CHUNK
ROWS_PER_W = ROWS // NW
GROUP = 8
NGROUPS = ROWS_PER_W // GROUP
GROUP_ROWS = GROUP * CHUNK

CBLOCKS = (VOCAB + 127) // 128
VPAD = CBLOCKS * 128
CJ = ((CBLOCKS + NW - 1) // NW + 1) // 2 * 2


def _make_relayout():
  mesh = plsc.VectorSubcoreMesh(core_axis_name="c", subcore_axis_name="s")

  @functools.partial(
      pl.kernel,
      out_type=jax.ShapeDtypeStruct((VPAD * D,), jnp.float32),
      mesh=mesh,
      scratch_types=[
          pltpu.VMEM((D, 128), jnp.float32),
          pltpu.VMEM((D, 128), jnp.float32),
          pltpu.VMEM((128 * D,), jnp.float32),
          pltpu.VMEM((128 * D,), jnp.float32),
          pltpu.SemaphoreType.DMA,
          pltpu.SemaphoreType.DMA,
      ],
      compiler_params=pltpu.CompilerParams(
          use_tc_tiling_on_sc=True, disable_bounds_checks=True,
          needs_layout_passes=False),
  )
  def relayout_kernel(tab_hbm, r_hbm, inb0, inb1, outb0, outb1, isem, osem):
    wid = lax.axis_index("s") * 2 + lax.axis_index("c")
    iota32 = lax.iota(jnp.int32, L) * D
    bufs = ((inb0, outb0), (inb1, outb1))

    def do_block(q, inb, outb):
      pltpu.async_copy(
          tab_hbm.at[:, pl.ds(q * 128, 128)], inb, isem).wait()

      for d in range(D):
        vecs = [inb[d, pl.ds(k * L, L)] for k in range(8)]
        for k in range(8):
          plsc.store_scatter(outb, [iota32 + (k * L * D + d)], vecs[k])

      pltpu.async_copy(
          outb, r_hbm.at[pl.ds(q * 128 * D, 128 * D)], osem)

    def pair_body(p, carry):
      for sub in range(2):
        j = p * 2 + sub
        q = wid + NW * j
        inb, outb = bufs[sub]

        @pl.when(q < CBLOCKS)
        def _():
          @pl.when(j >= 2)
          def _():
            pltpu.make_async_copy(
                outb, r_hbm.at[pl.ds(0, 128 * D)], osem).wait()
          do_block(q, inb, outb)
      return carry

    lax.fori_loop(0, CJ // 2, pair_body, 0)
    for _ in range(2):
      pltpu.make_async_copy(
          outb0, r_hbm.at[pl.ds(0, 128 * D)], osem).wait()

  return relayout_kernel


def _make_gather():
  mesh = plsc.VectorSubcoreMesh(core_axis_name="c", subcore_axis_name="s")

  @functools.partial(
      pl.kernel,
      out_type=jax.ShapeDtypeStruct((TOTAL, D), jnp.float32),
      mesh=mesh,
      scratch_types=[
          pltpu.VMEM((ROWS_PER_W, CHUNK), jnp.int32),
          pltpu.VMEM((2, GROUP_ROWS, D), jnp.float32),
          pltpu.SemaphoreType.DMA,
          pltpu.SemaphoreType.DMA,
      ],
      compiler_params=pltpu.CompilerParams(use_tc_tiling_on_sc=False),
  )
  def gather_kernel(table_hbm, idx_hbm, out_hbm, idx_v, rows_v, gsem, osem):
    wid = lax.axis_index("s") * 2 + lax.axis_index("c")
    base = wid * ROWS_PER_W
    pltpu.sync_copy(idx_hbm.at[pl.ds(base, ROWS_PER_W)], idx_v)

    def group_body(g, carry):
      buf = lax.rem(g, 2)

      @pl.when(g >= 2)
      def _():
        pltpu.make_async_copy(
            rows_v.at[buf], out_hbm.at[pl.ds(0, GROUP_ROWS)], osem).wait()

      for b in range(GROUP):
        pltpu.async_copy(
            table_hbm.at[idx_v.at[g * GROUP + b]],
            rows_v.at[buf, pl.ds(b * CHUNK, CHUNK)], gsem)
      for b in range(GROUP):
        pltpu.make_async_copy(
            table_hbm.at[idx_v.at[g * GROUP + b]],
            rows_v.at[buf, pl.ds(b * CHUNK, CHUNK)], gsem).wait()

      pltpu.async_copy(
          rows_v.at[buf],
          out_hbm.at[pl.ds((base + g * GROUP) * CHUNK, GROUP_ROWS)], osem)
      return carry

    lax.fori_loop(0, NGROUPS, group_body, 0)
    for _ in range(2):
      pltpu.make_async_copy(
          rows_v.at[0], out_hbm.at[pl.ds(0, GROUP_ROWS)], osem).wait()

  return gather_kernel


def _make_tiler():
  mesh = plsc.VectorSubcoreMesh(core_axis_name="c", subcore_axis_name="s")
  nq_per_w = (HIST * (BATCH // 128)) // NW

  @functools.partial(
      pl.kernel,
      out_type=jax.ShapeDtypeStruct((HIST, D, BATCH), jnp.float32),
      mesh=mesh,
      scratch_types=[
          pltpu.VMEM((128 * D,), jnp.float32),
          pltpu.VMEM((128 * D,), jnp.float32),
          pltpu.VMEM((1, D, 128), jnp.float32),
          pltpu.VMEM((1, D, 128), jnp.float32),
          pltpu.SemaphoreType.DMA,
          pltpu.SemaphoreType.DMA,
      ],
      compiler_params=pltpu.CompilerParams(
          use_tc_tiling_on_sc=True, needs_layout_passes=False),
  )
  def tiler_kernel(flat_hbm, out_hbm, inb0, inb1, outb0, outb1, isem, osem):
    wid = lax.axis_index("s") * 2 + lax.axis_index("c")
    zeros = jnp.zeros((L,), jnp.int32)
    iota_lo = lax.iota(jnp.int32, L)
    iota_hi = iota_lo + L
    bufs = ((inb0, outb0), (inb1, outb1))

    def do_chunk(q, inb, outb):
      pltpu.async_copy(
          flat_hbm.at[pl.ds(q * 128 * D, 128 * D)], inb, isem).wait()
      for t0 in range(0, 128, 8):
        vecs = [inb[pl.ds((t0 + u) * D + half * L, L)]
                for u in range(8) for half in range(2)]
        for u in range(8):
          tcol = jnp.full((L,), t0 + u, jnp.int32)
          plsc.store_scatter(outb, [zeros, iota_lo, tcol], vecs[2 * u])
          plsc.store_scatter(outb, [zeros, iota_hi, tcol], vecs[2 * u + 1])
      h = q // 128
      cb = lax.rem(q, 128)
      pltpu.async_copy(
          outb, out_hbm.at[pl.ds(h, 1), :, pl.ds(cb * 128, 128)], osem)

    def pair_body(p, carry):
      for sub in range(2):
        i = p * 2 + sub
        inb, outb = bufs[sub]

        @pl.when(i >= 2)
        def _():
          pltpu.make_async_copy(
              outb, out_hbm.at[pl.ds(0, 1), :, pl.ds(0, 128)], osem).wait()
        do_chunk(wid * nq_per_w + i, inb, outb)
      return carry

    lax.fori_loop(0, nq_per_w // 2, pair_body, 0)
    for _ in range(2):
      pltpu.make_async_copy(
          outb0, out_hbm.at[pl.ds(0, 1), :, pl.ds(0, 128)], osem).wait()

  return tiler_kernel


_relayout = _make_relayout()
_gather = _make_gather()
_tiler = _make_tiler()


@jax.jit
def kernel(x, embeddings):
  tab_t = jnp.transpose(embeddings)
  r_flat = _relayout(tab_t)
  r2d = r_flat.reshape(VPAD, D)
  idx2d = jnp.transpose(x).astype(jnp.int32).reshape(ROWS, CHUNK)
  rows = _gather(r2d, idx2d)
  flat = rows.reshape(TOTAL * D)
  out_t = _tiler(flat)
  return jnp.transpose(out_t, (2, 0, 1))


# speedup vs baseline: 1.6473x; 1.2872x over previous
import functools

import jax
import jax.numpy as jnp
from jax import lax
from jax.experimental import pallas as pl
from jax.experimental.pallas import tpu as pltpu
from jax.experimental.pallas import tpu_sc as plsc

VOCAB = 1000000
D = 32
BATCH = 16384
HIST = 50
L = 16
NW = 32

TOTAL = BATCH * HIST
CHUNK = 128
ROWS = TOTAL // CHUNK
ROWS_PER_W = ROWS // NW
GROUP = 8
NGROUPS = ROWS_PER_W // GROUP
GROUP_ROWS = GROUP * CHUNK

CBLOCKS = (VOCAB + 127) // 128
VPAD = CBLOCKS * 128
CJ = ((CBLOCKS + NW - 1) // NW + 3) // 4 * 4
BLK = 128 * D


def _make_relayout():
  mesh = plsc.VectorSubcoreMesh(core_axis_name="c", subcore_axis_name="s")

  @functools.partial(
      pl.kernel,
      out_type=jax.ShapeDtypeStruct((VPAD * D,), jnp.float32),
      mesh=mesh,
      scratch_types=(
          [pltpu.VMEM((D, 128), jnp.float32)] * 4
          + [pltpu.VMEM((BLK,), jnp.float32)] * 2
          + [pltpu.SemaphoreType.DMA] * 6
      ),
      compiler_params=pltpu.CompilerParams(
          use_tc_tiling_on_sc=True, disable_bounds_checks=True,
          needs_layout_passes=False),
  )
  def relayout_kernel(tab_hbm, r_hbm, in0, in1, in2, in3, ob0, ob1,
                      is0, is1, is2, is3, os0, os1):
    wid = lax.axis_index("s") * 2 + lax.axis_index("c")
    iota32 = lax.iota(jnp.int32, L) * D
    inbufs = ((in0, is0), (in1, is1), (in2, is2), (in3, is3))
    outbufs = ((ob0, os0), (ob1, os1))

    def qof(j):
      return wid + NW * j

    def start_in(j, slot):
      inb, isem = inbufs[slot]

      @pl.when(qof(j) < CBLOCKS)
      def _():
        pltpu.async_copy(
            tab_hbm.at[:, pl.ds(qof(j) * 128, 128)], inb, isem)

    for s in range(4):
      start_in(s, s)

    def quad_body(p, carry):
      for sub in range(4):
        j = p * 4 + sub
        q = qof(j)
        inb, isem = inbufs[sub]
        outb, osem = outbufs[sub % 2]

        @pl.when(q < CBLOCKS)
        def _():
          pltpu.make_async_copy(
              tab_hbm.at[:, pl.ds(0, 128)], inb, isem).wait()
          @pl.when(j >= 2)
          def _():
            pltpu.make_async_copy(
                outb, r_hbm.at[pl.ds(0, BLK)], osem).wait()
          for d in range(D):
            vecs = [inb[d, pl.ds(k * L, L)] for k in range(8)]
            for k in range(8):
              plsc.store_scatter(outb, [iota32 + (k * L * D + d)], vecs[k])
          start_in(j + 4, sub)
          pltpu.async_copy(outb, r_hbm.at[pl.ds(q * BLK, BLK)], osem)
      return carry

    lax.fori_loop(0, CJ // 4, quad_body, 0)
    for s in range(2):
      outb, osem = outbufs[s]
      pltpu.make_async_copy(outb, r_hbm.at[pl.ds(0, BLK)], osem).wait()

  return relayout_kernel


def _make_gather():
  mesh = plsc.VectorSubcoreMesh(core_axis_name="c", subcore_axis_name="s")

  @functools.partial(
      pl.kernel,
      out_type=jax.ShapeDtypeStruct((TOTAL, D), jnp.float32),
      mesh=mesh,
      scratch_types=[
          pltpu.VMEM((ROWS_PER_W, CHUNK), jnp.int32),
          pltpu.VMEM((2, GROUP_ROWS, D), jnp.float32),
          pltpu.SemaphoreType.DMA,
          pltpu.SemaphoreType.DMA,
          pltpu.SemaphoreType.DMA,
      ],
      compiler_params=pltpu.CompilerParams(use_tc_tiling_on_sc=False),
  )
  def gather_kernel(table_hbm, idx_hbm, out_hbm, idx_v, rows_v,
                    gsem, os0, os1):
    wid = lax.axis_index("s") * 2 + lax.axis_index("c")
    base = wid * ROWS_PER_W
    osems = (os0, os1)
    pltpu.sync_copy(idx_hbm.at[pl.ds(base, ROWS_PER_W)], idx_v)

    def group_body(g, carry):
      buf = lax.rem(g, 2)
      osem = None

      @pl.when(g >= 2)
      def _():
        for s in range(2):
          @pl.when(lax.rem(g, 2) == s)
          def _():
            pltpu.make_async_copy(
                rows_v.at[s], out_hbm.at[pl.ds(0, GROUP_ROWS)],
                osems[s]).wait()

      for b in range(GROUP):
        pltpu.async_copy(
            table_hbm.at[idx_v.at[g * GROUP + b]],
            rows_v.at[buf, pl.ds(b * CHUNK, CHUNK)], gsem)
      for b in range(GROUP):
        pltpu.make_async_copy(
            table_hbm.at[idx_v.at[g * GROUP + b]],
            rows_v.at[buf, pl.ds(b * CHUNK, CHUNK)], gsem).wait()

      for s in range(2):
        @pl.when(lax.rem(g, 2) == s)
        def _():
          pltpu.async_copy(
              rows_v.at[s],
              out_hbm.at[pl.ds((base + g * GROUP) * CHUNK, GROUP_ROWS)],
              osems[s])
      return carry

    lax.fori_loop(0, NGROUPS, group_body, 0)
    for s in range(2):
      pltpu.make_async_copy(
          rows_v.at[s], out_hbm.at[pl.ds(0, GROUP_ROWS)], osems[s]).wait()

  return gather_kernel


def _make_tiler():
  mesh = plsc.VectorSubcoreMesh(core_axis_name="c", subcore_axis_name="s")
  nq_per_w = (HIST * (BATCH // 128)) // NW

  @functools.partial(
      pl.kernel,
      out_type=jax.ShapeDtypeStruct((HIST, D, BATCH), jnp.float32),
      mesh=mesh,
      scratch_types=(
          [pltpu.VMEM((BLK,), jnp.float32)] * 4
          + [pltpu.VMEM((1, D, 128), jnp.float32)] * 2
          + [pltpu.SemaphoreType.DMA] * 6
      ),
      compiler_params=pltpu.CompilerParams(
          use_tc_tiling_on_sc=True, needs_layout_passes=False),
  )
  def tiler_kernel(flat_hbm, out_hbm, in0, in1, in2, in3, ob0, ob1,
                   is0, is1, is2, is3, os0, os1):
    wid = lax.axis_index("s") * 2 + lax.axis_index("c")
    zeros = jnp.zeros((L,), jnp.int32)
    iota_lo = lax.iota(jnp.int32, L)
    iota_hi = iota_lo + L
    inbufs = ((in0, is0), (in1, is1), (in2, is2), (in3, is3))
    outbufs = ((ob0, os0), (ob1, os1))
    qbase = wid * nq_per_w

    def start_in(i, slot):
      inb, isem = inbufs[slot]
      pltpu.async_copy(
          flat_hbm.at[pl.ds((qbase + i) * BLK, BLK)], inb, isem)

    for s in range(4):
      start_in(s, s)

    def quad_body(p, carry):
      for sub in range(4):
        i = p * 4 + sub
        q = qbase + i
        inb, isem = inbufs[sub]
        outb, osem = outbufs[sub % 2]

        pltpu.make_async_copy(
            flat_hbm.at[pl.ds(0, BLK)], inb, isem).wait()

        @pl.when(i >= 2)
        def _():
          pltpu.make_async_copy(
              outb, out_hbm.at[pl.ds(0, 1), :, pl.ds(0, 128)], osem).wait()

        for t0 in range(0, 128, 8):
          vecs = [inb[pl.ds((t0 + u) * D + half * L, L)]
                  for u in range(8) for half in range(2)]
          for u in range(8):
            tcol = jnp.full((L,), t0 + u, jnp.int32)
            plsc.store_scatter(outb, [zeros, iota_lo, tcol], vecs[2 * u])
            plsc.store_scatter(outb, [zeros, iota_hi, tcol], vecs[2 * u + 1])

        @pl.when(i + 4 < nq_per_w)
        def _():
          start_in(i + 4, sub)

        h = q // 128
        cb = lax.rem(q, 128)
        pltpu.async_copy(
            outb, out_hbm.at[pl.ds(h, 1), :, pl.ds(cb * 128, 128)], osem)
      return carry

    lax.fori_loop(0, nq_per_w // 4, quad_body, 0)
    for s in range(2):
      outb, osem = outbufs[s]
      pltpu.make_async_copy(
          outb, out_hbm.at[pl.ds(0, 1), :, pl.ds(0, 128)], osem).wait()

  return tiler_kernel


_relayout = _make_relayout()
_gather = _make_gather()
_tiler = _make_tiler()


@jax.jit
def kernel(x, embeddings):
  tab_t = jnp.transpose(embeddings)
  r_flat = _relayout(tab_t)
  r2d = r_flat.reshape(VPAD, D)
  idx2d = jnp.transpose(x).astype(jnp.int32).reshape(ROWS, CHUNK)
  rows = _gather(r2d, idx2d)
  flat = rows.reshape(TOTAL * D)
  out_t = _tiler(flat)
  return jnp.transpose(out_t, (2, 0, 1))


# speedup vs baseline: 1.6488x; 1.0009x over previous
import functools

import jax
import jax.numpy as jnp
from jax import lax
from jax.experimental import pallas as pl
from jax.experimental.pallas import tpu as pltpu
from jax.experimental.pallas import tpu_sc as plsc

VOCAB = 1000000
D = 32
BATCH = 16384
HIST = 50
L = 16
NW = 32

TOTAL = BATCH * HIST
CHUNK = 128
ROWS = TOTAL // CHUNK
ROWS_PER_W = ROWS // NW
GROUP = 8
NGROUPS = ROWS_PER_W // GROUP
GROUP_ROWS = GROUP * CHUNK

CBLOCKS = (VOCAB + 127) // 128
VPAD = CBLOCKS * 128
CJ = ((CBLOCKS + NW - 1) // NW + 3) // 4 * 4
BLK = 128 * D


def _make_relayout():
  mesh = plsc.VectorSubcoreMesh(core_axis_name="c", subcore_axis_name="s")

  @functools.partial(
      pl.kernel,
      out_type=jax.ShapeDtypeStruct((VPAD * D,), jnp.float32),
      mesh=mesh,
      scratch_types=(
          [pltpu.VMEM((D, 128), jnp.float32)] * 4
          + [pltpu.VMEM((BLK,), jnp.float32)] * 2
          + [pltpu.SemaphoreType.DMA] * 6
      ),
      compiler_params=pltpu.CompilerParams(
          use_tc_tiling_on_sc=True, disable_bounds_checks=True,
          needs_layout_passes=False),
  )
  def relayout_kernel(tab_hbm, r_hbm, in0, in1, in2, in3, ob0, ob1,
                      is0, is1, is2, is3, os0, os1):
    wid = lax.axis_index("s") * 2 + lax.axis_index("c")
    iota32 = lax.iota(jnp.int32, L) * D
    inbufs = ((in0, is0), (in1, is1), (in2, is2), (in3, is3))
    outbufs = ((ob0, os0), (ob1, os1))

    def qof(j):
      return wid + NW * j

    def start_in(j, slot):
      inb, isem = inbufs[slot]

      @pl.when(qof(j) < CBLOCKS)
      def _():
        pltpu.async_copy(
            tab_hbm.at[:, pl.ds(qof(j) * 128, 128)], inb, isem)

    for s in range(4):
      start_in(s, s)

    def quad_body(p, carry):
      for sub in range(4):
        j = p * 4 + sub
        q = qof(j)
        inb, isem = inbufs[sub]
        outb, osem = outbufs[sub % 2]

        @pl.when(q < CBLOCKS)
        def _():
          pltpu.make_async_copy(
              tab_hbm.at[:, pl.ds(0, 128)], inb, isem).wait()
          @pl.when(j >= 2)
          def _():
            pltpu.make_async_copy(
                outb, r_hbm.at[pl.ds(0, BLK)], osem).wait()
          for d in range(D):
            vecs = [inb[d, pl.ds(k * L, L)] for k in range(8)]
            for k in range(8):
              plsc.store_scatter(outb, [iota32 + (k * L * D + d)], vecs[k])
          start_in(j + 4, sub)
          pltpu.async_copy(outb, r_hbm.at[pl.ds(q * BLK, BLK)], osem)
      return carry

    lax.fori_loop(0, CJ // 4, quad_body, 0)
    for s in range(2):
      outb, osem = outbufs[s]
      pltpu.make_async_copy(outb, r_hbm.at[pl.ds(0, BLK)], osem).wait()

  return relayout_kernel


def _make_gather():
  mesh = plsc.VectorSubcoreMesh(core_axis_name="c", subcore_axis_name="s")

  @functools.partial(
      pl.kernel,
      out_type=jax.ShapeDtypeStruct((TOTAL, D), jnp.float32),
      mesh=mesh,
      scratch_types=[
          pltpu.VMEM((ROWS_PER_W, CHUNK), jnp.int32),
          pltpu.VMEM((2, GROUP_ROWS, D), jnp.float32),
          pltpu.SemaphoreType.DMA,
          pltpu.SemaphoreType.DMA,
          pltpu.SemaphoreType.DMA,
      ],
      compiler_params=pltpu.CompilerParams(use_tc_tiling_on_sc=False),
  )
  def gather_kernel(table_hbm, idx_hbm, out_hbm, idx_v, rows_v,
                    gsem, os0, os1):
    wid = lax.axis_index("s") * 2 + lax.axis_index("c")
    base = wid * ROWS_PER_W
    osems = (os0, os1)
    pltpu.sync_copy(idx_hbm.at[pl.ds(base, ROWS_PER_W)], idx_v)

    def group_body(g, carry):
      buf = lax.rem(g, 2)

      @pl.when(g >= 2)
      def _():
        for s in range(2):
          @pl.when(lax.rem(g, 2) == s)
          def _():
            pltpu.make_async_copy(
                rows_v.at[s], out_hbm.at[pl.ds(0, GROUP_ROWS)],
                osems[s]).wait()

      for b in range(GROUP):
        pltpu.async_copy(
            table_hbm.at[idx_v.at[g * GROUP + b]],
            rows_v.at[buf, pl.ds(b * CHUNK, CHUNK)], gsem)
      for b in range(GROUP):
        pltpu.make_async_copy(
            table_hbm.at[idx_v.at[g * GROUP + b]],
            rows_v.at[buf, pl.ds(b * CHUNK, CHUNK)], gsem).wait()

      for s in range(2):
        @pl.when(lax.rem(g, 2) == s)
        def _():
          pltpu.async_copy(
              rows_v.at[s],
              out_hbm.at[pl.ds((base + g * GROUP) * CHUNK, GROUP_ROWS)],
              osems[s])
      return carry

    lax.fori_loop(0, NGROUPS, group_body, 0)
    for s in range(2):
      pltpu.make_async_copy(
          rows_v.at[s], out_hbm.at[pl.ds(0, GROUP_ROWS)], osems[s]).wait()

  return gather_kernel


def _make_tiler():
  mesh = plsc.VectorSubcoreMesh(core_axis_name="c", subcore_axis_name="s")
  nq_per_w = (HIST * (BATCH // 128)) // NW

  @functools.partial(
      pl.kernel,
      out_type=jax.ShapeDtypeStruct((HIST, D, BATCH), jnp.float32),
      mesh=mesh,
      scratch_types=(
          [pltpu.VMEM((BLK,), jnp.float32)] * 4
          + [pltpu.VMEM((1, D, 128), jnp.float32)] * 2
          + [pltpu.SemaphoreType.DMA] * 6
      ),
      compiler_params=pltpu.CompilerParams(
          use_tc_tiling_on_sc=True, needs_layout_passes=False),
  )
  def tiler_kernel(flat_hbm, out_hbm, in0, in1, in2, in3, ob0, ob1,
                   is0, is1, is2, is3, os0, os1):
    wid = lax.axis_index("s") * 2 + lax.axis_index("c")
    zeros = jnp.zeros((L,), jnp.int32)
    iota_lo = lax.iota(jnp.int32, L)
    iota_hi = iota_lo + L
    inbufs = ((in0, is0), (in1, is1), (in2, is2), (in3, is3))
    outbufs = ((ob0, os0), (ob1, os1))
    qbase = wid * nq_per_w

    def start_in(i, slot):
      inb, isem = inbufs[slot]
      pltpu.async_copy(
          flat_hbm.at[pl.ds((qbase + i) * BLK, BLK)], inb, isem)

    for s in range(4):
      start_in(s, s)

    def quad_body(p, carry):
      for sub in range(4):
        i = p * 4 + sub
        q = qbase + i
        inb, isem = inbufs[sub]
        outb, osem = outbufs[sub % 2]

        pltpu.make_async_copy(
            flat_hbm.at[pl.ds(0, BLK)], inb, isem).wait()

        @pl.when(i >= 2)
        def _():
          pltpu.make_async_copy(
              outb, out_hbm.at[pl.ds(0, 1), :, pl.ds(0, 128)], osem).wait()

        for t0 in range(0, 128, 8):
          vecs = [inb[pl.ds((t0 + u) * D + half * L, L)]
                  for u in range(8) for half in range(2)]
          for u in range(8):
            tcol = jnp.full((L,), t0 + u, jnp.int32)
            plsc.store_scatter(outb, [zeros, iota_lo, tcol], vecs[2 * u])
            plsc.store_scatter(outb, [zeros, iota_hi, tcol], vecs[2 * u + 1])

        @pl.when(i + 4 < nq_per_w)
        def _():
          start_in(i + 4, sub)

        h = q // 128
        cb = lax.rem(q, 128)
        pltpu.async_copy(
            outb, out_hbm.at[pl.ds(h, 1), :, pl.ds(cb * 128, 128)], osem)
      return carry

    lax.fori_loop(0, nq_per_w // 4, quad_body, 0)
    for s in range(2):
      outb, osem = outbufs[s]
      pltpu.make_async_copy(
          outb, out_hbm.at[pl.ds(0, 1), :, pl.ds(0, 128)], osem).wait()

  return tiler_kernel


_relayout = _make_relayout()
_gather = _make_gather()
_tiler = _make_tiler()


@jax.jit
def kernel(x, embeddings):
  tab_t = jnp.transpose(embeddings)
  r_flat = _relayout(tab_t)
  r2d = r_flat.reshape(VPAD, D)
  idx2d = jnp.transpose(x).astype(jnp.int32).reshape(ROWS, CHUNK)
  rows = _gather(r2d, idx2d)
  flat = rows.reshape(TOTAL * D)
  out_t = _tiler(flat)
  return jnp.transpose(out_t, (2, 0, 1))
